# Initial kernel scaffold; baseline (speedup 1.0000x reference)
import jax
import jax.numpy as jnp
from jax.experimental import pallas as pl


def kernel(pij, i, j, z):
    raise NotImplementedError("write your pallas kernel here")


# speedup vs baseline: 5.9970x; 5.9970x over previous
import functools

import jax
import jax.numpy as jnp
from jax import lax
from jax.experimental import pallas as pl
from jax.experimental.pallas import tpu as pltpu
from jax.experimental.pallas import tpu_sc as plsc

_N_POINTS = 100000
_N_DIM = 64
_B = 262144
_NC = 2
_NS = 16
_L = 16
_NW = _NC * _NS
_CHUNK = _B // _NW
_G = 512
_NSUB = _CHUNK // _G
_NG = _G // _L

_LN2 = 0.6931471805599453
_EPS = 1e-12


def _ln(x):
    bits = lax.bitcast_convert_type(x, jnp.int32)
    e = ((bits >> 23) & 0xFF) - 127
    m = lax.bitcast_convert_type((bits & 0x007FFFFF) | 0x3F800000,
                                 jnp.float32)
    big = m > 1.4142135
    m = jnp.where(big, m * 0.5, m)
    ef = e.astype(jnp.float32) + jnp.where(big, 1.0, 0.0)
    s = (m - 1.0) / (m + 1.0)
    s2 = s * s
    t = 1.0 / 7.0 + s2 * (1.0 / 9.0)
    t = 1.0 / 5.0 + s2 * t
    t = 1.0 / 3.0 + s2 * t
    return ef * _LN2 + 2.0 * s * (1.0 + s2 * t)


@functools.cache
def _build_sc_partials():
    mesh = plsc.VectorSubcoreMesh(core_axis_name="c", subcore_axis_name="s")
    return pl.kernel(
        _sc_partials_body,
        mesh=mesh,
        compiler_params=pltpu.CompilerParams(use_tc_tiling_on_sc=False),
        out_type=jax.ShapeDtypeStruct((_NW, 4, _L), jnp.float32),
        scratch_types=[
            pltpu.VMEM((_G,), jnp.int32),
            pltpu.VMEM((_G,), jnp.int32),
            pltpu.VMEM((_G,), jnp.float32),
            pltpu.VMEM((_G, _N_DIM), jnp.float32),
            pltpu.VMEM((_G, _N_DIM), jnp.float32),
            pltpu.VMEM((4, _L), jnp.float32),
            pltpu.SemaphoreType.DMA,
            pltpu.SemaphoreType.DMA,
        ],
    )


def _sc_partials_body(pij_hbm, i_hbm, j_hbm, z_hbm, out_hbm,
                      ii_v, jj_v, p_v, zi_v, zj_v, acc_v, sem_a, sem_b):
    wid = lax.axis_index("s") * _NC + lax.axis_index("c")
    base = wid * _CHUNK
    lanes = lax.iota(jnp.int32, _L)
    zero = jnp.zeros((_L,), jnp.float32)

    def sub(t, carry):
        off = base + t * _G
        pltpu.sync_copy(i_hbm.at[pl.ds(off, _G)], ii_v)
        pltpu.sync_copy(j_hbm.at[pl.ds(off, _G)], jj_v)
        pltpu.sync_copy(pij_hbm.at[pl.ds(off, _G)], p_v)
        ca = pltpu.async_copy(z_hbm.at[ii_v], zi_v, sem_a)
        cb = pltpu.async_copy(z_hbm.at[jj_v], zj_v, sem_b)
        ca.wait()
        cb.wait()

        def group(g, carry2):
            qm1a, spqa, sppa, spa = carry2
            d = zero
            for p16 in range(_L):
                p = g * _L + p16
                acc = zero
                for c in range(_N_DIM // _L):
                    a = zi_v[p, pl.ds(c * _L, _L)]
                    b = zj_v[p, pl.ds(c * _L, _L)]
                    df = a - b
                    acc = acc + df * df
                for k in (1, 2, 4, 8):
                    acc = acc + acc.at[lanes ^ k].get(
                        mode="promise_in_bounds")
                d = jnp.where(lanes == p16, acc, d)
            q = 1.0 / (1.0 + d)
            p = p_v[pl.ds(g * _L, _L)]
            return (qm1a + (-d) * q,
                    spqa + p * _ln(q),
                    sppa + p * _ln(p + _EPS),
                    spa + p)

        return lax.fori_loop(0, _NG, group, carry)

    qm1a, spqa, sppa, spa = lax.fori_loop(
        0, _NSUB, sub, (zero, zero, zero, zero))
    acc_v[0] = qm1a
    acc_v[1] = spqa
    acc_v[2] = sppa
    acc_v[3] = spa
    pltpu.sync_copy(acc_v, out_hbm.at[wid])


def kernel(pij, i, j, z):
    parts = _build_sc_partials()(pij, i.astype(jnp.int32),
                                 j.astype(jnp.int32), z)
    qm1 = jnp.sum(parts[:, 0, :])
    spq = jnp.sum(parts[:, 1, :])
    spp = jnp.sum(parts[:, 2, :])
    sp = jnp.sum(parts[:, 3, :])
    s_total = jnp.float32(_B) + qm1
    return spp - spq + jnp.log(s_total) * sp


# speedup vs baseline: 8.8764x; 1.4801x over previous
import functools

import jax
import jax.numpy as jnp
from jax import lax
from jax.experimental import pallas as pl
from jax.experimental.pallas import tpu as pltpu
from jax.experimental.pallas import tpu_sc as plsc

_N_POINTS = 100000
_N_DIM = 64
_B = 262144
_NC = 2
_NS = 16
_L = 16
_NW = _NC * _NS
_CHUNK = _B // _NW
_G = 256
_NSUB = _CHUNK // _G
_NG = _G // _L

_LN2 = 0.6931471805599453
_EPS = 1e-12


def _ln(x):
    bits = lax.bitcast_convert_type(x, jnp.int32)
    e = ((bits >> 23) & 0xFF) - 127
    m = lax.bitcast_convert_type((bits & 0x007FFFFF) | 0x3F800000,
                                 jnp.float32)
    big = m > 1.4142135
    m = jnp.where(big, m * 0.5, m)
    ef = e.astype(jnp.float32) + jnp.where(big, 1.0, 0.0)
    s = (m - 1.0) / (m + 1.0)
    s2 = s * s
    t = 1.0 / 7.0 + s2 * (1.0 / 9.0)
    t = 1.0 / 5.0 + s2 * t
    t = 1.0 / 3.0 + s2 * t
    return ef * _LN2 + 2.0 * s * (1.0 + s2 * t)


@functools.cache
def _build_sc_partials():
    mesh = plsc.VectorSubcoreMesh(core_axis_name="c", subcore_axis_name="s")
    return pl.kernel(
        _sc_partials_body,
        mesh=mesh,
        compiler_params=pltpu.CompilerParams(use_tc_tiling_on_sc=False),
        out_type=jax.ShapeDtypeStruct((_NW, 4, _L), jnp.float32),
        scratch_types=[
            pltpu.VMEM((_CHUNK,), jnp.int32),
            pltpu.VMEM((_CHUNK,), jnp.int32),
            pltpu.VMEM((_CHUNK,), jnp.float32),
            pltpu.VMEM((_G, _N_DIM), jnp.float32),
            pltpu.VMEM((_G, _N_DIM), jnp.float32),
            pltpu.VMEM((_G, _N_DIM), jnp.float32),
            pltpu.VMEM((_G, _N_DIM), jnp.float32),
            pltpu.VMEM((4, _L), jnp.float32),
            pltpu.SemaphoreType.DMA,
            pltpu.SemaphoreType.DMA,
            pltpu.SemaphoreType.DMA,
            pltpu.SemaphoreType.DMA,
        ],
    )


def _sc_partials_body(pij_hbm, i_hbm, j_hbm, z_hbm, out_hbm,
                      ii_v, jj_v, p_v, zi0_v, zi1_v, zj0_v, zj1_v, acc_v,
                      si0, si1, sj0, sj1):
    wid = lax.axis_index("s") * _NC + lax.axis_index("c")
    base = wid * _CHUNK
    lanes = lax.iota(jnp.int32, _L)
    zero = jnp.zeros((_L,), jnp.float32)
    zbufs = ((zi0_v, zj0_v, si0, sj0), (zi1_v, zj1_v, si1, sj1))

    pltpu.sync_copy(i_hbm.at[pl.ds(base, _CHUNK)], ii_v)
    pltpu.sync_copy(j_hbm.at[pl.ds(base, _CHUNK)], jj_v)
    pltpu.sync_copy(pij_hbm.at[pl.ds(base, _CHUNK)], p_v)

    def copies(t, buf):
        zi, zj, szi, szj = zbufs[buf]
        ci = pltpu.make_async_copy(
            z_hbm.at[ii_v.at[pl.ds(t * _G, _G)]], zi, szi)
        cj = pltpu.make_async_copy(
            z_hbm.at[jj_v.at[pl.ds(t * _G, _G)]], zj, szj)
        return ci, cj

    ci0, cj0 = copies(0, 0)
    ci0.start()
    cj0.start()

    def merge(a, b, k):
        perm = lanes ^ k
        mask = (lanes & k) == 0
        a2 = a.at[perm].get(mode="promise_in_bounds")
        b2 = b.at[perm].get(mode="promise_in_bounds")
        return jnp.where(mask, a, b2) + jnp.where(mask, a2, b)

    def process(t, buf, carry):
        zi, zj, _, _ = zbufs[buf]

        def group(g, carry2):
            qm1a, spqa, sppa, spa = carry2

            def leaf(p16):
                p = g * _L + p16
                df = zi[p, pl.ds(0, _L)] - zj[p, pl.ds(0, _L)]
                acc = df * df
                for c in range(1, _N_DIM // _L):
                    df = zi[p, pl.ds(c * _L, _L)] - zj[p, pl.ds(c * _L, _L)]
                    acc = acc + df * df
                return acc

            def tree(lo, hi):
                if hi - lo == 1:
                    return leaf(lo)
                mid = (lo + hi) // 2
                return merge(tree(lo, mid), tree(mid, hi), mid - lo)

            d = tree(0, _L)
            q = 1.0 / (1.0 + d)
            p = p_v[pl.ds(t * _G + g * _L, _L)]
            return (qm1a + (-d) * q,
                    spqa + p * _ln(q),
                    sppa + p * _ln(p + _EPS),
                    spa + p)

        return lax.fori_loop(0, _NG, group, carry)

    def outer(tt, carry):
        for b in (0, 1):
            t = 2 * tt + b
            ci, cj = copies(t, b)
            ci.wait()
            cj.wait()

            @pl.when(t + 1 < _NSUB)
            def _():
                cin, cjn = copies(t + 1, 1 - b)
                cin.start()
                cjn.start()

            carry = process(t, b, carry)
        return carry

    qm1a, spqa, sppa, spa = lax.fori_loop(
        0, _NSUB // 2, outer, (zero, zero, zero, zero))
    acc_v[0] = qm1a
    acc_v[1] = spqa
    acc_v[2] = sppa
    acc_v[3] = spa
    pltpu.sync_copy(acc_v, out_hbm.at[wid])


def kernel(pij, i, j, z):
    parts = _build_sc_partials()(pij, i.astype(jnp.int32),
                                 j.astype(jnp.int32), z)
    qm1 = jnp.sum(parts[:, 0, :])
    spq = jnp.sum(parts[:, 1, :])
    spp = jnp.sum(parts[:, 2, :])
    sp = jnp.sum(parts[:, 3, :])
    s_total = jnp.float32(_B) + qm1
    return spp - spq + jnp.log(s_total) * sp


# speedup vs baseline: 12.4266x; 1.4000x over previous
import functools

import jax
import jax.numpy as jnp
from jax import lax
from jax.experimental import pallas as pl
from jax.experimental.pallas import tpu as pltpu
from jax.experimental.pallas import tpu_sc as plsc

_N_POINTS = 100000
_N_DIM = 64
_B = 262144
_NC = 2
_NS = 16
_L = 16
_NW = _NC * _NS
_CHUNK = _B // _NW
_G = 256
_NSUB = _CHUNK // _G
_NG = _G // _L

_LN2 = 0.6931471805599453
_EPS = 1e-12


def _ln(x):
    bits = lax.bitcast_convert_type(x, jnp.int32)
    e = ((bits >> 23) & 0xFF) - 127
    m = lax.bitcast_convert_type((bits & 0x007FFFFF) | 0x3F800000,
                                 jnp.float32)
    big = m > 1.4142135
    m = jnp.where(big, m * 0.5, m)
    ef = e.astype(jnp.float32) + jnp.where(big, 1.0, 0.0)
    s = (m - 1.0) / (m + 1.0)
    s2 = s * s
    t = 1.0 / 7.0 + s2 * (1.0 / 9.0)
    t = 1.0 / 5.0 + s2 * t
    t = 1.0 / 3.0 + s2 * t
    return ef * _LN2 + 2.0 * s * (1.0 + s2 * t)


@functools.cache
def _build_sc_partials():
    mesh = plsc.VectorSubcoreMesh(core_axis_name="c", subcore_axis_name="s")
    return pl.kernel(
        _sc_partials_body,
        mesh=mesh,
        compiler_params=pltpu.CompilerParams(use_tc_tiling_on_sc=False),
        out_type=jax.ShapeDtypeStruct((_NW, 4, _L), jnp.float32),
        scratch_types=[
            pltpu.VMEM((_CHUNK,), jnp.int32),
            pltpu.VMEM((_CHUNK,), jnp.int32),
            pltpu.VMEM((_CHUNK,), jnp.float32),
            pltpu.VMEM((_G, _N_DIM), jnp.float32),
            pltpu.VMEM((_G, _N_DIM), jnp.float32),
            pltpu.VMEM((_G, _N_DIM), jnp.float32),
            pltpu.VMEM((_G, _N_DIM), jnp.float32),
            pltpu.VMEM((_G,), jnp.float32),
            pltpu.VMEM((4, _L), jnp.float32),
            pltpu.SemaphoreType.DMA,
            pltpu.SemaphoreType.DMA,
            pltpu.SemaphoreType.DMA,
            pltpu.SemaphoreType.DMA,
        ],
    )


def _sc_partials_body(pij_hbm, i_hbm, j_hbm, z_hbm, out_hbm,
                      ii_v, jj_v, p_v, zi0_v, zi1_v, zj0_v, zj1_v, d_v,
                      acc_v, si0, si1, sj0, sj1):
    wid = lax.axis_index("s") * _NC + lax.axis_index("c")
    base = wid * _CHUNK
    lanes = lax.iota(jnp.int32, _L)
    zero = jnp.zeros((_L,), jnp.float32)
    zbufs = ((zi0_v, zj0_v, si0, sj0), (zi1_v, zj1_v, si1, sj1))

    pltpu.sync_copy(i_hbm.at[pl.ds(base, _CHUNK)], ii_v)
    pltpu.sync_copy(j_hbm.at[pl.ds(base, _CHUNK)], jj_v)
    pltpu.sync_copy(pij_hbm.at[pl.ds(base, _CHUNK)], p_v)

    def copies(t, buf):
        zi, zj, szi, szj = zbufs[buf]
        ci = pltpu.make_async_copy(
            z_hbm.at[ii_v.at[pl.ds(t * _G, _G)]], zi, szi)
        cj = pltpu.make_async_copy(
            z_hbm.at[jj_v.at[pl.ds(t * _G, _G)]], zj, szj)
        return ci, cj

    ci0, cj0 = copies(0, 0)
    ci0.start()
    cj0.start()

    def merge(a, b, k):
        perm = lanes ^ k
        mask = (lanes & k) == 0
        a2 = a.at[perm].get(mode="promise_in_bounds")
        b2 = b.at[perm].get(mode="promise_in_bounds")
        return jnp.where(mask, a, b2) + jnp.where(mask, a2, b)

    def process(t, buf, carry):
        zi, zj, _, _ = zbufs[buf]

        def dist_group(g, c0):
            def leaf(p16):
                p = g * _L + p16
                df = zi[p, pl.ds(0, _L)] - zj[p, pl.ds(0, _L)]
                acc = df * df
                for c in range(1, _N_DIM // _L):
                    df = zi[p, pl.ds(c * _L, _L)] - zj[p, pl.ds(c * _L, _L)]
                    acc = acc + df * df
                return acc

            def tree(lo, hi):
                if hi - lo == 1:
                    return leaf(lo)
                mid = (lo + hi) // 2
                return merge(tree(lo, mid), tree(mid, hi), mid - lo)

            d_v[pl.ds(g * _L, _L)] = tree(0, _L)
            return c0

        lax.fori_loop(0, _NG, dist_group, 0)

        def acc_group(g, carry2):
            qm1a, spqa, sppa, spa = carry2
            d = d_v[pl.ds(g * _L, _L)]
            q = 1.0 / (1.0 + d)
            p = p_v[pl.ds(t * _G + g * _L, _L)]
            return (qm1a + (-d) * q,
                    spqa + p * _ln(q),
                    sppa + p * _ln(p + _EPS),
                    spa + p)

        return lax.fori_loop(0, _NG, acc_group, carry)

    def outer(tt, carry):
        for b in (0, 1):
            t = 2 * tt + b
            ci, cj = copies(t, b)
            ci.wait()
            cj.wait()

            @pl.when(t + 1 < _NSUB)
            def _():
                cin, cjn = copies(t + 1, 1 - b)
                cin.start()
                cjn.start()

            carry = process(t, b, carry)
        return carry

    qm1a, spqa, sppa, spa = lax.fori_loop(
        0, _NSUB // 2, outer, (zero, zero, zero, zero))
    acc_v[0] = qm1a
    acc_v[1] = spqa
    acc_v[2] = sppa
    acc_v[3] = spa
    pltpu.sync_copy(acc_v, out_hbm.at[wid])


def kernel(pij, i, j, z):
    parts = _build_sc_partials()(pij, i.astype(jnp.int32),
                                 j.astype(jnp.int32), z)
    qm1 = jnp.sum(parts[:, 0, :])
    spq = jnp.sum(parts[:, 1, :])
    spp = jnp.sum(parts[:, 2, :])
    sp = jnp.sum(parts[:, 3, :])
    s_total = jnp.float32(_B) + qm1
    return spp - spq + jnp.log(s_total) * sp
